# baseline (speedup 1.0000x reference)
import functools

import jax
import jax.numpy as jnp
from jax import lax
from jax.experimental import pallas as pl
from jax.experimental.pallas import tpu as pltpu
from jax.experimental.pallas import tpu_sc as plsc

N = 10000
E = 320000
D = 128
NG = 16
NC = 2
NS = 16
NW = NC * NS
EP = E // NW
ECH = 100
NCH = EP // ECH
NPT = N // NS
NRW = 632
NPAD = 10240
DPT = NPAD // NS

RB = 2000
NBLK = N // RB


@functools.lru_cache(maxsize=None)
def _sc_mesh():
  return plsc.VectorSubcoreMesh(
      core_axis_name="c", subcore_axis_name="s", num_cores=NC, num_subcores=NS)


@functools.lru_cache(maxsize=None)
def _deg_kernel():
  return pl.kernel(
      _deg_body,
      out_type=jax.ShapeDtypeStruct((NC, NPAD), jnp.float32),
      mesh=_sc_mesh(),
      scratch_types=[
          pltpu.VMEM((NCH, ECH), jnp.int32),
          pltpu.VMEM((DPT,), jnp.float32),
          pltpu.VMEM((DPT,), jnp.float32),
          pltpu.VMEM_SHARED((NPAD,), jnp.float32)
      ],
  )


def _deg_call(dst3):
  return _deg_kernel()(dst3)


def _deg_body(dst_hbm, out_hbm, didx_v, ones_v, init_v, deg_sh):
  cid = lax.axis_index("c")
  sid = lax.axis_index("s")
  wid = sid * NC + cid
  pltpu.sync_copy(dst_hbm.at[wid], didx_v)
  one = jnp.ones((16,), jnp.float32)
  seed = one * jnp.where(cid == 0, 1.0, 0.0).astype(jnp.float32)

  @pl.loop(0, DPT // 16)
  def _fill(i):
    ones_v[pl.ds(i * 16, 16)] = one
    init_v[pl.ds(i * 16, 16)] = seed

  pltpu.sync_copy(init_v, deg_sh.at[pl.ds(sid * DPT, DPT)])
  plsc.subcore_barrier()

  @pl.loop(0, NCH)
  def _scatter(j):
    pltpu.sync_copy(ones_v.at[pl.ds(0, ECH)], deg_sh.at[didx_v.at[j]],
                    add=True)

  plsc.subcore_barrier()
  pltpu.sync_copy(deg_sh.at[pl.ds(sid * DPT, DPT)],
                  out_hbm.at[cid, pl.ds(sid * DPT, DPT)])


@functools.lru_cache(maxsize=None)
def _prop_kernel():
  return pl.kernel(
      _prop_body,
      out_type=jax.ShapeDtypeStruct((NC, N, D), jnp.float32),
      mesh=_sc_mesh(),
      scratch_types=[
          pltpu.VMEM((NCH, ECH), jnp.int32),
          pltpu.VMEM((NCH, ECH), jnp.int32),
          pltpu.VMEM((ECH, D), jnp.float32),
          pltpu.VMEM((ECH, D), jnp.float32),
          pltpu.VMEM_SHARED((N, D), jnp.float32),
          pltpu.SemaphoreType.DMA,
          pltpu.SemaphoreType.DMA,
      ],
  )


def _prop_call(table, src3, dst3):
  return _prop_kernel()(table, src3, dst3)


def _prop_body(table_hbm, src_hbm, dst_hbm, out_hbm, sidx_v, didx_v, rows_a,
               rows_b, acc_sh, sem_a, sem_b):
  cid = lax.axis_index("c")
  sid = lax.axis_index("s")
  wid = sid * NC + cid
  pltpu.sync_copy(src_hbm.at[wid], sidx_v)
  pltpu.sync_copy(dst_hbm.at[wid], didx_v)
  row0 = pl.multiple_of(sid * NPT - lax.rem(sid, 8), 8)
  pltpu.sync_copy(table_hbm.at[pl.ds(row0, NRW)],
                  acc_sh.at[pl.ds(row0, NRW)])
  plsc.subcore_barrier()

  @pl.loop(0, NCH)
  def _edges(j):
    pltpu.sync_copy(table_hbm.at[sidx_v.at[j]], rows_a)
    pltpu.sync_copy(rows_a, acc_sh.at[didx_v.at[j]], add=True)

  plsc.subcore_barrier()
  pltpu.sync_copy(acc_sh.at[pl.ds(row0, NRW)],
                  out_hbm.at[cid, pl.ds(row0, NRW)])


def _prep_body(degp_ref, x_ref, dinv_ref, xs_ref):
  deg = degp_ref[:, 0:1] + degp_ref[:, 1:2]
  dinv = lax.rsqrt(deg)
  dinv_ref[...] = dinv
  xs_ref[...] = x_ref[...] * dinv


def _prep_call(degp_t, x):
  return pl.pallas_call(
      _prep_body,
      grid=(NBLK,),
      in_specs=[
          pl.BlockSpec((RB, 2), lambda i: (i, 0)),
          pl.BlockSpec((RB, D), lambda i: (i, 0)),
      ],
      out_specs=[
          pl.BlockSpec((RB, 1), lambda i: (i, 0)),
          pl.BlockSpec((RB, D), lambda i: (i, 0)),
      ],
      out_shape=[
          jax.ShapeDtypeStruct((N, 1), jnp.float32),
          jax.ShapeDtypeStruct((N, D), jnp.float32),
      ],
  )(degp_t, x)


def _mid_body(acc_ref, xs_ref, dinv_ref, w1_ref, b1_ref, hs_ref):
  e = acc_ref[0] + acc_ref[1] - xs_ref[...]
  p = e * dinv_ref[...]
  h = jnp.dot(p, w1_ref[...], preferred_element_type=jnp.float32)
  h = jnp.maximum(h + b1_ref[...], 0.0)
  hs_ref[...] = h * dinv_ref[...]


def _mid_call(acc, xs, dinv, w1, b1):
  return pl.pallas_call(
      _mid_body,
      grid=(NBLK,),
      in_specs=[
          pl.BlockSpec((NC, RB, D), lambda i: (0, i, 0)),
          pl.BlockSpec((RB, D), lambda i: (i, 0)),
          pl.BlockSpec((RB, 1), lambda i: (i, 0)),
          pl.BlockSpec((D, D), lambda i: (0, 0)),
          pl.BlockSpec((1, D), lambda i: (0, 0)),
      ],
      out_specs=pl.BlockSpec((RB, D), lambda i: (i, 0)),
      out_shape=jax.ShapeDtypeStruct((N, D), jnp.float32),
  )(acc, xs, dinv, w1, b1)


def _final_body(acc_ref, hs_ref, dinv_ref, batch_ref, w2_ref, b2_ref,
                ddi_ref, wg1_ref, bg1_ref, wg2_ref, bg2_ref,
                wf1_ref, bf1_ref, wf2_ref, bf2_ref, wf3_ref, bf3_ref,
                pool_ref, out_ref):
  i = pl.program_id(0)
  e = acc_ref[0] + acc_ref[1] - hs_ref[...]
  p = e * dinv_ref[...]
  h2 = jnp.dot(p, w2_ref[...], preferred_element_type=jnp.float32)
  h2 = jnp.maximum(h2 + b2_ref[...], 0.0)

  neg = jnp.float32(-jnp.inf)

  @pl.when(i == 0)
  def _init():
    pool_ref[...] = jnp.full((NG, 2 * D), neg, jnp.float32)
    out_ref[...] = jnp.zeros((NG, 33), jnp.float32)

  b = batch_ref[...]
  for g in range(NG):
    cand = jnp.max(jnp.where(b == g, h2, neg), axis=0, keepdims=True)
    pool_ref[pl.ds(g, 1), :] = jnp.maximum(pool_ref[pl.ds(g, 1), :], cand)

  @pl.when(i == NBLK - 1)
  def _mlp():
    pooled = pool_ref[...]
    g1 = jnp.dot(pooled, wg1_ref[...], preferred_element_type=jnp.float32)
    g1 = jnp.maximum(g1 + bg1_ref[...], 0.0)
    g2 = jnp.dot(g1, wg2_ref[...], preferred_element_type=jnp.float32)
    g2 = g2 + bg2_ref[...]
    d1 = jnp.dot(ddi_ref[...], wf1_ref[...],
                 preferred_element_type=jnp.float32)
    d1 = jnp.maximum(d1 + bf1_ref[...], 0.0)
    d2 = jnp.dot(d1, wf2_ref[...], preferred_element_type=jnp.float32)
    d2 = jnp.maximum(d2 + bf2_ref[...], 0.0)
    d3 = jnp.dot(d2, wf3_ref[...], preferred_element_type=jnp.float32)
    d3 = jnp.maximum(d3 + bf3_ref[...], 0.0)
    out_ref[:, 0:32] = g2
    out_ref[:, 32:33] = d3


def _final_call(acc, hs, dinv, batch2, w2, b2, ddi, wg1, bg1, wg2, bg2,
                wf1, bf1, wf2, bf2, wf3, bf3):
  full = lambda shape: pl.BlockSpec(shape, lambda i: tuple(0 for _ in shape))
  _, out = pl.pallas_call(
      _final_body,
      grid=(NBLK,),
      in_specs=[
          pl.BlockSpec((NC, RB, D), lambda i: (0, i, 0)),
          pl.BlockSpec((RB, D), lambda i: (i, 0)),
          pl.BlockSpec((RB, 1), lambda i: (i, 0)),
          pl.BlockSpec((RB, 1), lambda i: (i, 0)),
          full((D, 2 * D)),
          full((1, 2 * D)),
          full((NG, D)),
          full((2 * D, 1028)),
          full((1, 1028)),
          full((1028, 32)),
          full((1, 32)),
          full((D, 64)),
          full((1, 64)),
          full((64, NG)),
          full((1, NG)),
          full((NG, 1)),
          full((1, 1)),
      ],
      out_specs=[
          pl.BlockSpec((NG, 2 * D), lambda i: (0, 0)),
          pl.BlockSpec((NG, 33), lambda i: (0, 0)),
      ],
      out_shape=[
          jax.ShapeDtypeStruct((NG, 2 * D), jnp.float32),
          jax.ShapeDtypeStruct((NG, 33), jnp.float32),
      ],
  )(acc, hs, dinv, batch2, w2, b2, ddi, wg1, bg1, wg2, bg2,
    wf1, bf1, wf2, bf2, wf3, bf3)
  return out


def kernel(x, edge_index, batch, DDI_features, protein_mask,
           W1, b1, W2, b2, Wg1, bg1, Wg2, bg2,
           Wf1, bf1, Wf2, bf2, Wf3, bf3):
  src3 = edge_index[0].reshape(NW, NCH, ECH)
  dst3 = edge_index[1].reshape(NW, NCH, ECH)

  degp = _deg_call(dst3)
  degp_t = jnp.transpose(degp[:, :N])
  dinv, xs = _prep_call(degp_t, x)

  acc1 = _prop_call(xs, src3, dst3)
  hs = _mid_call(acc1, xs, dinv, W1, b1.reshape(1, D))

  acc2 = _prop_call(hs, src3, dst3)
  out = _final_call(acc2, hs, dinv, batch.reshape(N, 1), W2,
                    b2.reshape(1, 2 * D), DDI_features,
                    Wg1, bg1.reshape(1, 1028), Wg2, bg2.reshape(1, 32),
                    Wf1, bf1.reshape(1, 64), Wf2, bf2.reshape(1, NG),
                    Wf3, bf3.reshape(1, 1))
  return out


# speedup vs baseline: 1.0169x; 1.0169x over previous
import functools

import jax
import jax.numpy as jnp
from jax import lax
from jax.experimental import pallas as pl
from jax.experimental.pallas import tpu as pltpu
from jax.experimental.pallas import tpu_sc as plsc

N = 10000
E = 320000
D = 128
FH = D // 2
NG = 16
NC = 2
NS = 16
NW = NC * NS
ECH = 125
EPD = E // NW
NCD = EPD // ECH
EPP = E // NS
NCP = EPP // ECH
NPT = N // NS
NRW = 632
NPAD = 10240
DPT = NPAD // NS

RB = 2000
NBLK = N // RB


@functools.lru_cache(maxsize=None)
def _sc_mesh():
  return plsc.VectorSubcoreMesh(
      core_axis_name="c", subcore_axis_name="s", num_cores=NC, num_subcores=NS)


@functools.lru_cache(maxsize=None)
def _deg_kernel():
  return pl.kernel(
      _deg_body,
      out_type=jax.ShapeDtypeStruct((NC, NPAD), jnp.float32),
      mesh=_sc_mesh(),
      scratch_types=[
          pltpu.VMEM((NCD, ECH), jnp.int32),
          pltpu.VMEM((DPT,), jnp.float32),
          pltpu.VMEM((DPT,), jnp.float32),
          pltpu.VMEM_SHARED((NPAD,), jnp.float32)
      ],
  )


def _deg_call(dst3):
  return _deg_kernel()(dst3)


def _deg_body(dst_hbm, out_hbm, didx_v, ones_v, init_v, deg_sh):
  cid = lax.axis_index("c")
  sid = lax.axis_index("s")
  wid = sid * NC + cid
  pltpu.sync_copy(dst_hbm.at[wid], didx_v)
  one = jnp.ones((16,), jnp.float32)
  seed = one * jnp.where(cid == 0, 1.0, 0.0).astype(jnp.float32)

  @pl.loop(0, DPT // 16)
  def _fill(i):
    ones_v[pl.ds(i * 16, 16)] = one
    init_v[pl.ds(i * 16, 16)] = seed

  pltpu.sync_copy(init_v, deg_sh.at[pl.ds(sid * DPT, DPT)])
  plsc.subcore_barrier()

  @pl.loop(0, NCD)
  def _scatter(j):
    pltpu.sync_copy(ones_v.at[pl.ds(0, ECH)], deg_sh.at[didx_v.at[j]],
                    add=True)

  plsc.subcore_barrier()
  pltpu.sync_copy(deg_sh.at[pl.ds(sid * DPT, DPT)],
                  out_hbm.at[cid, pl.ds(sid * DPT, DPT)])


@functools.lru_cache(maxsize=None)
def _prop_kernel():
  return pl.kernel(
      _prop_body,
      out_type=jax.ShapeDtypeStruct((NC, N, FH), jnp.float32),
      mesh=_sc_mesh(),
      scratch_types=[
          pltpu.VMEM((NCP, ECH), jnp.int32),
          pltpu.VMEM((NCP, ECH), jnp.int32),
          pltpu.VMEM((ECH, FH), jnp.float32),
          pltpu.VMEM((ECH, FH), jnp.float32),
          pltpu.VMEM_SHARED((N, FH), jnp.float32),
          pltpu.SemaphoreType.DMA,
          pltpu.SemaphoreType.DMA,
      ],
      compiler_params=pltpu.CompilerParams(use_tc_tiling_on_sc=False),
  )


def _prop_call(table, src3, dst3):
  return _prop_kernel()(table, src3, dst3)


def _prop_body(table_hbm, src_hbm, dst_hbm, out_hbm, sidx_v, didx_v, rows_a,
               rows_b, acc_sh, sem_a, sem_b):
  cid = lax.axis_index("c")
  sid = lax.axis_index("s")
  pltpu.sync_copy(src_hbm.at[sid], sidx_v)
  pltpu.sync_copy(dst_hbm.at[sid], didx_v)
  row0 = pl.multiple_of(sid * NPT - lax.rem(sid, 8), 8)
  pltpu.sync_copy(table_hbm.at[cid, pl.ds(row0, NRW)],
                  acc_sh.at[pl.ds(row0, NRW)])
  plsc.subcore_barrier()

  pltpu.async_copy(table_hbm.at[cid].at[sidx_v.at[0]], rows_a, sem_a)

  @pl.loop(0, NCP // 2)
  def _edges(jj):
    j0 = jj * 2
    j1 = j0 + 1
    pltpu.make_async_copy(
        table_hbm.at[cid].at[sidx_v.at[j0]], rows_a, sem_a).wait()
    pltpu.async_copy(table_hbm.at[cid].at[sidx_v.at[j1]], rows_b, sem_b)
    pltpu.sync_copy(rows_a, acc_sh.at[didx_v.at[j0]], add=True)
    pltpu.make_async_copy(
        table_hbm.at[cid].at[sidx_v.at[j1]], rows_b, sem_b).wait()
    j2 = jnp.where(j1 + 1 < NCP, j1 + 1, 0)
    pltpu.async_copy(table_hbm.at[cid].at[sidx_v.at[j2]], rows_a, sem_a)
    pltpu.sync_copy(rows_b, acc_sh.at[didx_v.at[j1]], add=True)

  pltpu.make_async_copy(table_hbm.at[cid].at[sidx_v.at[0]], rows_a,
                        sem_a).wait()
  plsc.subcore_barrier()
  pltpu.sync_copy(acc_sh.at[pl.ds(row0, NRW)],
                  out_hbm.at[cid, pl.ds(row0, NRW)])


def _prep_body(degp_ref, x_ref, dinv_ref, xs_ref):
  deg = degp_ref[:, 0:1] + degp_ref[:, 1:2]
  dinv = lax.rsqrt(deg)
  dinv_ref[...] = dinv
  xs = x_ref[...] * dinv
  xs_ref[0] = xs[:, :FH]
  xs_ref[1] = xs[:, FH:]


def _prep_call(degp_t, x):
  return pl.pallas_call(
      _prep_body,
      grid=(NBLK,),
      in_specs=[
          pl.BlockSpec((RB, 2), lambda i: (i, 0)),
          pl.BlockSpec((RB, D), lambda i: (i, 0)),
      ],
      out_specs=[
          pl.BlockSpec((RB, 1), lambda i: (i, 0)),
          pl.BlockSpec((NC, RB, FH), lambda i: (0, i, 0)),
      ],
      out_shape=[
          jax.ShapeDtypeStruct((N, 1), jnp.float32),
          jax.ShapeDtypeStruct((NC, N, FH), jnp.float32),
      ],
  )(degp_t, x)


def _mid_body(acc_ref, dinv_ref, w1_ref, b1_ref, hs_ref):
  p = jnp.concatenate([acc_ref[0], acc_ref[1]], axis=1) * dinv_ref[...]
  h = jnp.dot(p, w1_ref[...], preferred_element_type=jnp.float32)
  h = jnp.maximum(h + b1_ref[...], 0.0)
  hs = h * dinv_ref[...]
  hs_ref[0] = hs[:, :FH]
  hs_ref[1] = hs[:, FH:]


def _mid_call(acc, dinv, w1, b1):
  return pl.pallas_call(
      _mid_body,
      grid=(NBLK,),
      in_specs=[
          pl.BlockSpec((NC, RB, FH), lambda i: (0, i, 0)),
          pl.BlockSpec((RB, 1), lambda i: (i, 0)),
          pl.BlockSpec((D, D), lambda i: (0, 0)),
          pl.BlockSpec((1, D), lambda i: (0, 0)),
      ],
      out_specs=pl.BlockSpec((NC, RB, FH), lambda i: (0, i, 0)),
      out_shape=jax.ShapeDtypeStruct((NC, N, FH), jnp.float32),
  )(acc, dinv, w1, b1)


def _final_body(acc_ref, dinv_ref, batch_ref, w2_ref, b2_ref,
                ddi_ref, wg1_ref, bg1_ref, wg2_ref, bg2_ref,
                wf1_ref, bf1_ref, wf2_ref, bf2_ref, wf3_ref, bf3_ref,
                pool_ref, out_ref):
  i = pl.program_id(0)
  p = jnp.concatenate([acc_ref[0], acc_ref[1]], axis=1) * dinv_ref[...]
  h2 = jnp.dot(p, w2_ref[...], preferred_element_type=jnp.float32)
  h2 = jnp.maximum(h2 + b2_ref[...], 0.0)

  neg = jnp.float32(-jnp.inf)

  @pl.when(i == 0)
  def _init():
    pool_ref[...] = jnp.full((NG, 2 * D), neg, jnp.float32)
    out_ref[...] = jnp.zeros((NG, 33), jnp.float32)

  b = batch_ref[...]
  for g in range(NG):
    cand = jnp.max(jnp.where(b == g, h2, neg), axis=0, keepdims=True)
    pool_ref[pl.ds(g, 1), :] = jnp.maximum(pool_ref[pl.ds(g, 1), :], cand)

  @pl.when(i == NBLK - 1)
  def _mlp():
    pooled = pool_ref[...]
    g1 = jnp.dot(pooled, wg1_ref[...], preferred_element_type=jnp.float32)
    g1 = jnp.maximum(g1 + bg1_ref[...], 0.0)
    g2 = jnp.dot(g1, wg2_ref[...], preferred_element_type=jnp.float32)
    g2 = g2 + bg2_ref[...]
    d1 = jnp.dot(ddi_ref[...], wf1_ref[...],
                 preferred_element_type=jnp.float32)
    d1 = jnp.maximum(d1 + bf1_ref[...], 0.0)
    d2 = jnp.dot(d1, wf2_ref[...], preferred_element_type=jnp.float32)
    d2 = jnp.maximum(d2 + bf2_ref[...], 0.0)
    d3 = jnp.dot(d2, wf3_ref[...], preferred_element_type=jnp.float32)
    d3 = jnp.maximum(d3 + bf3_ref[...], 0.0)
    out_ref[:, 0:32] = g2
    out_ref[:, 32:33] = d3


def _final_call(acc, dinv, batch2, w2, b2, ddi, wg1, bg1, wg2, bg2,
                wf1, bf1, wf2, bf2, wf3, bf3):
  full = lambda shape: pl.BlockSpec(shape, lambda i: tuple(0 for _ in shape))
  _, out = pl.pallas_call(
      _final_body,
      grid=(NBLK,),
      in_specs=[
          pl.BlockSpec((NC, RB, FH), lambda i: (0, i, 0)),
          pl.BlockSpec((RB, 1), lambda i: (i, 0)),
          pl.BlockSpec((RB, 1), lambda i: (i, 0)),
          full((D, 2 * D)),
          full((1, 2 * D)),
          full((NG, D)),
          full((2 * D, 1028)),
          full((1, 1028)),
          full((1028, 32)),
          full((1, 32)),
          full((D, 64)),
          full((1, 64)),
          full((64, NG)),
          full((1, NG)),
          full((NG, 1)),
          full((1, 1)),
      ],
      out_specs=[
          pl.BlockSpec((NG, 2 * D), lambda i: (0, 0)),
          pl.BlockSpec((NG, 33), lambda i: (0, 0)),
      ],
      out_shape=[
          jax.ShapeDtypeStruct((NG, 2 * D), jnp.float32),
          jax.ShapeDtypeStruct((NG, 33), jnp.float32),
      ],
  )(acc, dinv, batch2, w2, b2, ddi, wg1, bg1, wg2, bg2,
    wf1, bf1, wf2, bf2, wf3, bf3)
  return out


def kernel(x, edge_index, batch, DDI_features, protein_mask,
           W1, b1, W2, b2, Wg1, bg1, Wg2, bg2,
           Wf1, bf1, Wf2, bf2, Wf3, bf3):
  src3 = edge_index[0].reshape(NS, NCP, ECH)
  dst3 = edge_index[1].reshape(NS, NCP, ECH)
  dst3d = edge_index[1].reshape(NW, NCD, ECH)

  degp = _deg_call(dst3d)
  degp_t = jnp.transpose(degp[:, :N])
  dinv, xs = _prep_call(degp_t, x)

  acc1 = _prop_call(xs, src3, dst3)
  hs = _mid_call(acc1, dinv, W1, b1.reshape(1, D))

  acc2 = _prop_call(hs, src3, dst3)
  out = _final_call(acc2, dinv, batch.reshape(N, 1), W2,
                    b2.reshape(1, 2 * D), DDI_features,
                    Wg1, bg1.reshape(1, 1028), Wg2, bg2.reshape(1, 32),
                    Wf1, bf1.reshape(1, 64), Wf2, bf2.reshape(1, NG),
                    Wf3, bf3.reshape(1, 1))
  return out


# speedup vs baseline: 1.5002x; 1.4753x over previous
import functools

import jax
import jax.numpy as jnp
from jax import lax
from jax.experimental import pallas as pl
from jax.experimental.pallas import tpu as pltpu
from jax.experimental.pallas import tpu_sc as plsc

N = 10000
E = 320000
D = 128
FH = D // 2
NG = 16
NC = 2
NS = 16
NW = NC * NS
EP = E // NW
ECH = 125
NCH = EP // ECH
EPP = E // NS
NCP = EPP // ECH
NBUF = 4
NPT = N // NS
NRW = 632
NPAD = 10240
DPT = NPAD // NS

RB = 2000
NBLK = N // RB


@functools.lru_cache(maxsize=None)
def _sc_mesh():
  return plsc.VectorSubcoreMesh(
      core_axis_name="c", subcore_axis_name="s", num_cores=NC, num_subcores=NS)


@functools.lru_cache(maxsize=None)
def _deg_kernel():
  return pl.kernel(
      _deg_body,
      out_type=jax.ShapeDtypeStruct((NC, NPAD), jnp.float32),
      mesh=_sc_mesh(),
      scratch_types=[
          pltpu.VMEM((NCH, ECH), jnp.int32),
          pltpu.VMEM((DPT,), jnp.float32),
          pltpu.VMEM((DPT,), jnp.float32),
          pltpu.VMEM_SHARED((NPAD,), jnp.float32)
      ],
  )


def _deg_call(dst3):
  return _deg_kernel()(dst3)


def _deg_body(dst_hbm, out_hbm, didx_v, ones_v, init_v, deg_sh):
  cid = lax.axis_index("c")
  sid = lax.axis_index("s")
  wid = sid * NC + cid
  pltpu.sync_copy(dst_hbm.at[wid], didx_v)
  one = jnp.ones((16,), jnp.float32)
  seed = one * jnp.where(cid == 0, 1.0, 0.0).astype(jnp.float32)

  @pl.loop(0, DPT // 16)
  def _fill(i):
    ones_v[pl.ds(i * 16, 16)] = one
    init_v[pl.ds(i * 16, 16)] = seed

  pltpu.sync_copy(init_v, deg_sh.at[pl.ds(sid * DPT, DPT)])
  plsc.subcore_barrier()

  @pl.loop(0, NCH)
  def _scatter(j):
    pltpu.sync_copy(ones_v.at[pl.ds(0, ECH)], deg_sh.at[didx_v.at[j]],
                    add=True)

  plsc.subcore_barrier()
  pltpu.sync_copy(deg_sh.at[pl.ds(sid * DPT, DPT)],
                  out_hbm.at[cid, pl.ds(sid * DPT, DPT)])


@functools.lru_cache(maxsize=None)
def _prop_kernel():
  return pl.kernel(
      _prop_body,
      out_type=jax.ShapeDtypeStruct((NC, N, FH), jnp.float32),
      mesh=_sc_mesh(),
      scratch_types=[
          pltpu.VMEM((NCP, ECH), jnp.int32),
          pltpu.VMEM((NCP, ECH), jnp.int32),
          [pltpu.VMEM((ECH, FH), jnp.float32) for _ in range(NBUF)],
          pltpu.VMEM_SHARED((N, FH), jnp.float32),
          [pltpu.SemaphoreType.DMA for _ in range(NBUF)],
      ],
      compiler_params=pltpu.CompilerParams(use_tc_tiling_on_sc=False),
  )


def _prop_call(table, src3, dst3):
  return _prop_kernel()(table, src3, dst3)


def _prop_body(table_hbm, src_hbm, dst_hbm, out_hbm, sidx_v, didx_v, rows,
               acc_sh, sems):
  cid = lax.axis_index("c")
  sid = lax.axis_index("s")
  pltpu.sync_copy(src_hbm.at[sid], sidx_v)
  pltpu.sync_copy(dst_hbm.at[sid], didx_v)
  row0 = pl.multiple_of(sid * NPT - lax.rem(sid, 8), 8)
  pltpu.sync_copy(table_hbm.at[cid, pl.ds(row0, NRW)],
                  acc_sh.at[pl.ds(row0, NRW)])
  plsc.subcore_barrier()

  def gather(j, b):
    pltpu.async_copy(table_hbm.at[cid].at[sidx_v.at[j]], rows[b], sems[b])

  def wait(j, b):
    pltpu.make_async_copy(
        table_hbm.at[cid].at[sidx_v.at[j]], rows[b], sems[b]).wait()

  for b in range(NBUF):
    gather(b, b)

  @pl.loop(0, NCP // NBUF)
  def _edges(jj):
    j0 = jj * NBUF
    for b in range(NBUF):
      j = j0 + b
      wait(j, b)
      pltpu.sync_copy(rows[b], acc_sh.at[didx_v.at[j]], add=True)
      jn = j + NBUF
      jn = jnp.where(jn < NCP, jn, b)
      gather(jn, b)

  for b in range(NBUF):
    wait(b, b)
  plsc.subcore_barrier()
  pltpu.sync_copy(acc_sh.at[pl.ds(row0, NRW)],
                  out_hbm.at[cid, pl.ds(row0, NRW)])


def _prep_body(degp_ref, x_ref, dinv_ref, xs_ref):
  deg = degp_ref[:, 0:1] + degp_ref[:, 1:2]
  dinv = lax.rsqrt(deg)
  dinv_ref[...] = dinv
  xs = x_ref[...] * dinv
  xs_ref[0] = xs[:, :FH]
  xs_ref[1] = xs[:, FH:]


def _prep_call(degp_t, x):
  return pl.pallas_call(
      _prep_body,
      grid=(NBLK,),
      in_specs=[
          pl.BlockSpec((RB, 2), lambda i: (i, 0)),
          pl.BlockSpec((RB, D), lambda i: (i, 0)),
      ],
      out_specs=[
          pl.BlockSpec((RB, 1), lambda i: (i, 0)),
          pl.BlockSpec((NC, RB, FH), lambda i: (0, i, 0)),
      ],
      out_shape=[
          jax.ShapeDtypeStruct((N, 1), jnp.float32),
          jax.ShapeDtypeStruct((NC, N, FH), jnp.float32),
      ],
  )(degp_t, x)


def _mid_body(acc_ref, dinv_ref, w1_ref, b1_ref, hs_ref):
  p = jnp.concatenate([acc_ref[0], acc_ref[1]], axis=1) * dinv_ref[...]
  h = jnp.dot(p, w1_ref[...], preferred_element_type=jnp.float32)
  h = jnp.maximum(h + b1_ref[...], 0.0)
  hs = h * dinv_ref[...]
  hs_ref[0] = hs[:, :FH]
  hs_ref[1] = hs[:, FH:]


def _mid_call(acc, dinv, w1, b1):
  return pl.pallas_call(
      _mid_body,
      grid=(NBLK,),
      in_specs=[
          pl.BlockSpec((NC, RB, FH), lambda i: (0, i, 0)),
          pl.BlockSpec((RB, 1), lambda i: (i, 0)),
          pl.BlockSpec((D, D), lambda i: (0, 0)),
          pl.BlockSpec((1, D), lambda i: (0, 0)),
      ],
      out_specs=pl.BlockSpec((NC, RB, FH), lambda i: (0, i, 0)),
      out_shape=jax.ShapeDtypeStruct((NC, N, FH), jnp.float32),
  )(acc, dinv, w1, b1)


def _final_body(acc_ref, dinv_ref, batch_ref, w2_ref, b2_ref,
                ddi_ref, wg1_ref, bg1_ref, wg2_ref, bg2_ref,
                wf1_ref, bf1_ref, wf2_ref, bf2_ref, wf3_ref, bf3_ref,
                pool_ref, out_ref):
  i = pl.program_id(0)
  p = jnp.concatenate([acc_ref[0], acc_ref[1]], axis=1) * dinv_ref[...]
  h2 = jnp.dot(p, w2_ref[...], preferred_element_type=jnp.float32)
  h2 = jnp.maximum(h2 + b2_ref[...], 0.0)

  neg = jnp.float32(-jnp.inf)

  @pl.when(i == 0)
  def _init():
    pool_ref[...] = jnp.full((NG, 2 * D), neg, jnp.float32)
    out_ref[...] = jnp.zeros((NG, 33), jnp.float32)

  b = batch_ref[...]
  for g in range(NG):
    cand = jnp.max(jnp.where(b == g, h2, neg), axis=0, keepdims=True)
    pool_ref[pl.ds(g, 1), :] = jnp.maximum(pool_ref[pl.ds(g, 1), :], cand)

  @pl.when(i == NBLK - 1)
  def _mlp():
    pooled = pool_ref[...]
    g1 = jnp.dot(pooled, wg1_ref[...], preferred_element_type=jnp.float32)
    g1 = jnp.maximum(g1 + bg1_ref[...], 0.0)
    g2 = jnp.dot(g1, wg2_ref[...], preferred_element_type=jnp.float32)
    g2 = g2 + bg2_ref[...]
    d1 = jnp.dot(ddi_ref[...], wf1_ref[...],
                 preferred_element_type=jnp.float32)
    d1 = jnp.maximum(d1 + bf1_ref[...], 0.0)
    d2 = jnp.dot(d1, wf2_ref[...], preferred_element_type=jnp.float32)
    d2 = jnp.maximum(d2 + bf2_ref[...], 0.0)
    d3 = jnp.dot(d2, wf3_ref[...], preferred_element_type=jnp.float32)
    d3 = jnp.maximum(d3 + bf3_ref[...], 0.0)
    out_ref[:, 0:32] = g2
    out_ref[:, 32:33] = d3


def _final_call(acc, dinv, batch2, w2, b2, ddi, wg1, bg1, wg2, bg2,
                wf1, bf1, wf2, bf2, wf3, bf3):
  full = lambda shape: pl.BlockSpec(shape, lambda i: tuple(0 for _ in shape))
  _, out = pl.pallas_call(
      _final_body,
      grid=(NBLK,),
      in_specs=[
          pl.BlockSpec((NC, RB, FH), lambda i: (0, i, 0)),
          pl.BlockSpec((RB, 1), lambda i: (i, 0)),
          pl.BlockSpec((RB, 1), lambda i: (i, 0)),
          full((D, 2 * D)),
          full((1, 2 * D)),
          full((NG, D)),
          full((2 * D, 1028)),
          full((1, 1028)),
          full((1028, 32)),
          full((1, 32)),
          full((D, 64)),
          full((1, 64)),
          full((64, NG)),
          full((1, NG)),
          full((NG, 1)),
          full((1, 1)),
      ],
      out_specs=[
          pl.BlockSpec((NG, 2 * D), lambda i: (0, 0)),
          pl.BlockSpec((NG, 33), lambda i: (0, 0)),
      ],
      out_shape=[
          jax.ShapeDtypeStruct((NG, 2 * D), jnp.float32),
          jax.ShapeDtypeStruct((NG, 33), jnp.float32),
      ],
  )(acc, dinv, batch2, w2, b2, ddi, wg1, bg1, wg2, bg2,
    wf1, bf1, wf2, bf2, wf3, bf3)
  return out


def kernel(x, edge_index, batch, DDI_features, protein_mask,
           W1, b1, W2, b2, Wg1, bg1, Wg2, bg2,
           Wf1, bf1, Wf2, bf2, Wf3, bf3):
  src3 = edge_index[0].reshape(NS, NCP, ECH)
  dst3 = edge_index[1].reshape(NS, NCP, ECH)
  dst3d = edge_index[1].reshape(NW, NCH, ECH)

  degp = _deg_call(dst3d)
  degp_t = jnp.transpose(degp[:, :N])
  dinv, xs = _prep_call(degp_t, x)

  acc1 = _prop_call(xs, src3, dst3)
  hs = _mid_call(acc1, dinv, W1, b1.reshape(1, D))

  acc2 = _prop_call(hs, src3, dst3)
  out = _final_call(acc2, dinv, batch.reshape(N, 1), W2,
                    b2.reshape(1, 2 * D), DDI_features,
                    Wg1, bg1.reshape(1, 1028), Wg2, bg2.reshape(1, 32),
                    Wf1, bf1.reshape(1, 64), Wf2, bf2.reshape(1, NG),
                    Wf3, bf3.reshape(1, 1))
  return out


# speedup vs baseline: 1.6006x; 1.0669x over previous
import functools

import jax
import jax.numpy as jnp
from jax import lax
from jax.experimental import pallas as pl
from jax.experimental.pallas import tpu as pltpu
from jax.experimental.pallas import tpu_sc as plsc

N = 10000
E = 320000
D = 128
FH = D // 2
NG = 16
NC = 2
NS = 16
NW = NC * NS
ECH = 128
ER = E // ECH
NCD = 72
NCP = 152
NBUF = 4
NPT = N // NS
NRW = 632
NPAD = 10240
DPT = NPAD // NS

RB = 2000
NBLK = N // RB


@functools.lru_cache(maxsize=None)
def _sc_mesh():
  return plsc.VectorSubcoreMesh(
      core_axis_name="c", subcore_axis_name="s", num_cores=NC, num_subcores=NS)


@functools.lru_cache(maxsize=None)
def _deg_kernel():
  return pl.kernel(
      _deg_body,
      out_type=jax.ShapeDtypeStruct((NC, NPAD), jnp.float32),
      mesh=_sc_mesh(),
      scratch_types=[
          pltpu.VMEM((NCD + 8, ECH), jnp.int32),
          pltpu.VMEM((DPT,), jnp.float32),
          pltpu.VMEM((DPT,), jnp.float32),
          pltpu.VMEM_SHARED((NPAD,), jnp.float32)
      ],
  )


def _deg_call(dst3):
  return _deg_kernel()(dst3)


def _deg_body(dst_hbm, out_hbm, didx_v, ones_v, init_v, deg_sh):
  cid = lax.axis_index("c")
  sid = lax.axis_index("s")
  wid = sid * NC + cid
  pltpu.sync_copy(dst_hbm.at[pl.ds(wid * NCD, NCD)],
                  didx_v.at[pl.ds(0, NCD)])
  base = NW * NCD
  nch = NCD + jnp.where(wid < 24, 8, 0) + jnp.where(wid == 24, 4, 0)

  @pl.when(wid < 24)
  def _extra_idx():
    ofs = pl.multiple_of(base + wid * 8, 8)
    pltpu.sync_copy(dst_hbm.at[pl.ds(ofs, 8)], didx_v.at[pl.ds(NCD, 8)])

  @pl.when(wid == 24)
  def _last_idx():
    pltpu.sync_copy(dst_hbm.at[pl.ds(2496, 4)], didx_v.at[pl.ds(NCD, 4)])

  one = jnp.ones((16,), jnp.float32)
  seed = one * jnp.where(cid == 0, 1.0, 0.0).astype(jnp.float32)

  @pl.loop(0, DPT // 16)
  def _fill(i):
    ones_v[pl.ds(i * 16, 16)] = one
    init_v[pl.ds(i * 16, 16)] = seed

  pltpu.sync_copy(init_v, deg_sh.at[pl.ds(sid * DPT, DPT)])
  plsc.subcore_barrier()

  @pl.loop(0, nch)
  def _scatter(j):
    pltpu.sync_copy(ones_v.at[pl.ds(0, ECH)], deg_sh.at[didx_v.at[j]],
                    add=True)

  plsc.subcore_barrier()
  pltpu.sync_copy(deg_sh.at[pl.ds(sid * DPT, DPT)],
                  out_hbm.at[cid, pl.ds(sid * DPT, DPT)])


@functools.lru_cache(maxsize=None)
def _prop_kernel():
  return pl.kernel(
      _prop_body,
      out_type=jax.ShapeDtypeStruct((N, D), jnp.float32),
      mesh=_sc_mesh(),
      scratch_types=[
          pltpu.VMEM((NCP + 8, ECH), jnp.int32),
          pltpu.VMEM((NCP + 8, ECH), jnp.int32),
          [pltpu.VMEM((ECH, FH), jnp.float32) for _ in range(NBUF)],
          pltpu.VMEM_SHARED((N, FH), jnp.float32),
          [pltpu.SemaphoreType.DMA for _ in range(NBUF)],
      ],
      compiler_params=pltpu.CompilerParams(use_tc_tiling_on_sc=False),
  )


def _prop_call(table, src3, dst3):
  return _prop_kernel()(table, src3, dst3)


def _prop_body(table_hbm, src_hbm, dst_hbm, out_hbm, sidx_v, didx_v, rows,
               acc_sh, sems):
  cid = lax.axis_index("c")
  sid = lax.axis_index("s")
  pltpu.sync_copy(src_hbm.at[pl.ds(sid * NCP, NCP)],
                  sidx_v.at[pl.ds(0, NCP)])
  pltpu.sync_copy(dst_hbm.at[pl.ds(sid * NCP, NCP)],
                  didx_v.at[pl.ds(0, NCP)])
  base = NS * NCP
  nch = NCP + jnp.where(sid < 8, 8, 0) + jnp.where(sid == 8, 4, 0)

  @pl.when(sid < 8)
  def _extra_idx():
    ofs = pl.multiple_of(base + sid * 8, 8)
    pltpu.sync_copy(src_hbm.at[pl.ds(ofs, 8)], sidx_v.at[pl.ds(NCP, 8)])
    pltpu.sync_copy(dst_hbm.at[pl.ds(ofs, 8)], didx_v.at[pl.ds(NCP, 8)])

  @pl.when(sid == 8)
  def _last_idx():
    pltpu.sync_copy(src_hbm.at[pl.ds(2496, 4)], sidx_v.at[pl.ds(NCP, 4)])
    pltpu.sync_copy(dst_hbm.at[pl.ds(2496, 4)], didx_v.at[pl.ds(NCP, 4)])

  row0 = pl.multiple_of(sid * NPT - lax.rem(sid, 8), 8)
  pltpu.sync_copy(table_hbm.at[cid, pl.ds(row0, NRW)],
                  acc_sh.at[pl.ds(row0, NRW)])
  plsc.subcore_barrier()

  def gather(j, b):
    pltpu.async_copy(table_hbm.at[cid].at[sidx_v.at[j]], rows[b], sems[b])

  def wait(j, b):
    pltpu.make_async_copy(
        table_hbm.at[cid].at[sidx_v.at[j]], rows[b], sems[b]).wait()

  for b in range(NBUF):
    gather(b, b)

  @pl.loop(0, nch // NBUF)
  def _edges(jj):
    j0 = jj * NBUF
    for b in range(NBUF):
      j = j0 + b
      wait(j, b)
      pltpu.sync_copy(rows[b], acc_sh.at[didx_v.at[j]], add=True)
      jn = j + NBUF
      jn = jnp.where(jn < nch, jn, b)
      gather(jn, b)

  for b in range(NBUF):
    wait(b, b)

  plsc.subcore_barrier()
  pltpu.sync_copy(acc_sh.at[pl.ds(row0, NRW)],
                  out_hbm.at[pl.ds(row0, NRW), pl.ds(cid * FH, FH)])


def _prep_body(degp_ref, x_ref, dinv_ref, xs_ref):
  deg = degp_ref[:, 0:1] + degp_ref[:, 1:2]
  dinv = lax.rsqrt(deg)
  dinv_ref[...] = dinv
  xs = x_ref[...] * dinv
  xs_ref[0] = xs[:, :FH]
  xs_ref[1] = xs[:, FH:]


def _prep_call(degp_t, x):
  return pl.pallas_call(
      _prep_body,
      grid=(NBLK,),
      in_specs=[
          pl.BlockSpec((RB, 2), lambda i: (i, 0)),
          pl.BlockSpec((RB, D), lambda i: (i, 0)),
      ],
      out_specs=[
          pl.BlockSpec((RB, 1), lambda i: (i, 0)),
          pl.BlockSpec((NC, RB, FH), lambda i: (0, i, 0)),
      ],
      out_shape=[
          jax.ShapeDtypeStruct((N, 1), jnp.float32),
          jax.ShapeDtypeStruct((NC, N, FH), jnp.float32),
      ],
  )(degp_t, x)


def _mid_body(acc_ref, dinv_ref, w1_ref, b1_ref, hs_ref):
  p = acc_ref[...] * dinv_ref[...]
  h = jnp.dot(p, w1_ref[...], preferred_element_type=jnp.float32)
  h = jnp.maximum(h + b1_ref[...], 0.0)
  hs = h * dinv_ref[...]
  hs_ref[0] = hs[:, :FH]
  hs_ref[1] = hs[:, FH:]


def _mid_call(acc, dinv, w1, b1):
  return pl.pallas_call(
      _mid_body,
      grid=(NBLK,),
      in_specs=[
          pl.BlockSpec((RB, D), lambda i: (i, 0)),
          pl.BlockSpec((RB, 1), lambda i: (i, 0)),
          pl.BlockSpec((D, D), lambda i: (0, 0)),
          pl.BlockSpec((1, D), lambda i: (0, 0)),
      ],
      out_specs=pl.BlockSpec((NC, RB, FH), lambda i: (0, i, 0)),
      out_shape=jax.ShapeDtypeStruct((NC, N, FH), jnp.float32),
  )(acc, dinv, w1, b1)


def _final_body(acc_ref, dinv_ref, batch_ref, w2_ref, b2_ref,
                ddi_ref, wg1_ref, bg1_ref, wg2_ref, bg2_ref,
                wf1_ref, bf1_ref, wf2_ref, bf2_ref, wf3_ref, bf3_ref,
                pool_ref, out_ref):
  i = pl.program_id(0)
  p = acc_ref[...] * dinv_ref[...]
  h2 = jnp.dot(p, w2_ref[...], preferred_element_type=jnp.float32)
  h2 = jnp.maximum(h2 + b2_ref[...], 0.0)

  neg = jnp.float32(-jnp.inf)

  @pl.when(i == 0)
  def _init():
    pool_ref[...] = jnp.full((NG, 2 * D), neg, jnp.float32)
    out_ref[...] = jnp.zeros((NG, 33), jnp.float32)

  b = batch_ref[...]
  for g in range(NG):
    cand = jnp.max(jnp.where(b == g, h2, neg), axis=0, keepdims=True)
    pool_ref[pl.ds(g, 1), :] = jnp.maximum(pool_ref[pl.ds(g, 1), :], cand)

  @pl.when(i == NBLK - 1)
  def _mlp():
    pooled = pool_ref[...]
    g1 = jnp.dot(pooled, wg1_ref[...], preferred_element_type=jnp.float32)
    g1 = jnp.maximum(g1 + bg1_ref[...], 0.0)
    g2 = jnp.dot(g1, wg2_ref[...], preferred_element_type=jnp.float32)
    g2 = g2 + bg2_ref[...]
    d1 = jnp.dot(ddi_ref[...], wf1_ref[...],
                 preferred_element_type=jnp.float32)
    d1 = jnp.maximum(d1 + bf1_ref[...], 0.0)
    d2 = jnp.dot(d1, wf2_ref[...], preferred_element_type=jnp.float32)
    d2 = jnp.maximum(d2 + bf2_ref[...], 0.0)
    d3 = jnp.dot(d2, wf3_ref[...], preferred_element_type=jnp.float32)
    d3 = jnp.maximum(d3 + bf3_ref[...], 0.0)
    out_ref[:, 0:32] = g2
    out_ref[:, 32:33] = d3


def _final_call(acc, dinv, batch2, w2, b2, ddi, wg1, bg1, wg2, bg2,
                wf1, bf1, wf2, bf2, wf3, bf3):
  full = lambda shape: pl.BlockSpec(shape, lambda i: tuple(0 for _ in shape))
  _, out = pl.pallas_call(
      _final_body,
      grid=(NBLK,),
      in_specs=[
          pl.BlockSpec((RB, D), lambda i: (i, 0)),
          pl.BlockSpec((RB, 1), lambda i: (i, 0)),
          pl.BlockSpec((RB, 1), lambda i: (i, 0)),
          full((D, 2 * D)),
          full((1, 2 * D)),
          full((NG, D)),
          full((2 * D, 1028)),
          full((1, 1028)),
          full((1028, 32)),
          full((1, 32)),
          full((D, 64)),
          full((1, 64)),
          full((64, NG)),
          full((1, NG)),
          full((NG, 1)),
          full((1, 1)),
      ],
      out_specs=[
          pl.BlockSpec((NG, 2 * D), lambda i: (0, 0)),
          pl.BlockSpec((NG, 33), lambda i: (0, 0)),
      ],
      out_shape=[
          jax.ShapeDtypeStruct((NG, 2 * D), jnp.float32),
          jax.ShapeDtypeStruct((NG, 33), jnp.float32),
      ],
  )(acc, dinv, batch2, w2, b2, ddi, wg1, bg1, wg2, bg2,
    wf1, bf1, wf2, bf2, wf3, bf3)
  return out


def kernel(x, edge_index, batch, DDI_features, protein_mask,
           W1, b1, W2, b2, Wg1, bg1, Wg2, bg2,
           Wf1, bf1, Wf2, bf2, Wf3, bf3):
  src2 = edge_index[0].reshape(ER, ECH)
  dst2 = edge_index[1].reshape(ER, ECH)

  degp = _deg_call(dst2)
  degp_t = jnp.transpose(degp[:, :N])
  dinv, xs = _prep_call(degp_t, x)

  acc1 = _prop_call(xs, src2, dst2)
  hs = _mid_call(acc1, dinv, W1, b1.reshape(1, D))

  acc2 = _prop_call(hs, src2, dst2)
  out = _final_call(acc2, dinv, batch.reshape(N, 1), W2,
                    b2.reshape(1, 2 * D), DDI_features,
                    Wg1, bg1.reshape(1, 1028), Wg2, bg2.reshape(1, 32),
                    Wf1, bf1.reshape(1, 64), Wf2, bf2.reshape(1, NG),
                    Wf3, bf3.reshape(1, 1))
  return out
